# baseline (speedup 1.0000x reference)
import functools

import jax
import jax.numpy as jnp
from jax import lax
from jax.experimental import pallas as pl
from jax.experimental.pallas import tpu as pltpu
from jax.experimental.pallas import tpu_sc as plsc


def _tc_body(hs_ref, gum_ref, w_ref, b_ref, perp_ref, idx_ref, *, B, TL, G, V, gmin):
    D1 = hs_ref.shape[2]
    hs = hs_ref[...].reshape(B * TL, D1)
    logits = jnp.dot(hs, w_ref[...], preferred_element_type=jnp.float32)
    logits = logits + b_ref[...][None, :]
    perp_parts = []
    idx_parts = []
    for g in range(G):
        lg = logits[:, g * V:(g + 1) * V].reshape(B, TL, V)
        a = lg + gum_ref[:, :, g * V:(g + 1) * V]
        am = jnp.max(a, axis=-1, keepdims=True)
        ii = lax.broadcasted_iota(jnp.int32, (B, TL, V), 2)
        idx = jnp.min(jnp.where(a == am, ii, jnp.int32(2 ** 30)), axis=-1)
        idx_parts.append(idx + g * V)
        e = jnp.exp(lg - (am - gmin))
        s = jnp.sum(e, axis=-1, keepdims=True)
        p = jnp.sum(e / s, axis=0) * (1.0 / B)
        perp_parts.append(p[:, None, :])
    perp_ref[...] = jnp.concatenate(perp_parts, axis=1)
    idx_ref[...] = jnp.concatenate(idx_parts, axis=0)


def _tc_call(hs, gum, W, b2, TL):
    B, L, D1 = hs.shape
    GV = W.shape[1]
    G = 2
    V = GV // G
    grid = (L // TL,)
    return pl.pallas_call(
        functools.partial(_tc_body, B=B, TL=TL, G=G, V=V, gmin=-10.0),
        grid=grid,
        in_specs=[
            pl.BlockSpec((B, TL, D1), lambda i: (0, i, 0)),
            pl.BlockSpec((B, TL, GV), lambda i: (0, i, 0)),
            pl.BlockSpec((D1, GV), lambda i: (0, 0)),
            pl.BlockSpec((GV,), lambda i: (0,)),
        ],
        out_specs=[
            pl.BlockSpec((TL, G, V), lambda i: (i, 0, 0)),
            pl.BlockSpec((G * B, TL), lambda i: (0, i)),
        ],
        out_shape=[
            jax.ShapeDtypeStruct((L, G, V), jnp.float32),
            jax.ShapeDtypeStruct((G * B, L), jnp.int32),
        ],
        compiler_params=pltpu.CompilerParams(
            dimension_semantics=("parallel",)),
    )(hs, gum, W, b2)


def _sc_gather(table, gidx, B, L):
    info = plsc.get_sparse_core_info()
    NC, NS = info.num_cores, info.num_subcores
    NW = NC * NS
    CH = gidx.shape[1]
    DC = table.shape[1]
    G = NW * CH * 128 // (B * L)
    lw = L // (NW // B)
    wpb = NW // B
    mesh = plsc.VectorSubcoreMesh(core_axis_name="c", subcore_axis_name="s")

    @functools.partial(
        pl.kernel,
        mesh=mesh,
        out_type=jax.ShapeDtypeStruct((B, L, G * DC), jnp.float32),
        scratch_types=[
            pltpu.VMEM((CH, 128), jnp.int32),
            pltpu.VMEM((CH, 128, DC), jnp.float32),
        ] + [pltpu.SemaphoreType.DMA] * 5,
    )
    def k(table_hbm, idx_hbm, out_hbm, idx_v, rows_v, *sems):
        wid = lax.axis_index("s") * NC + lax.axis_index("c")
        wb = wid // wpb
        lworker = (wid % wpb) * lw
        lchunk = lw // CH
        pltpu.sync_copy(idx_hbm.at[wid], idx_v)
        gcopies = [
            pltpu.async_copy(table_hbm.at[idx_v.at[j]], rows_v.at[j], sems[j])
            for j in range(CH)
        ]
        wcopies = []
        for j in range(CH):
            gcopies[j].wait()
            wcopies.append(pltpu.async_copy(
                rows_v.at[j].reshape(lchunk, G * DC),
                out_hbm.at[wb, pl.ds(lworker + j * lchunk, lchunk), :],
                sems[CH]))
        for c in wcopies:
            c.wait()

    return k(table, gidx)


def kernel(hidden_states, W, b, code_book):
    B, L, D1 = hidden_states.shape
    _, G, V, DC = code_book.shape
    with jax.ensure_compile_time_eval():
        gum = jax.random.gumbel(jax.random.key(42), (B, L, G * V), jnp.float32)
    TL = 512
    perp, idx = _tc_call(hidden_states, gum, W, b, TL)
    info = plsc.get_sparse_core_info()
    NW = info.num_cores * info.num_subcores
    with jax.ensure_compile_time_eval():
        p = jnp.arange(B * L * G, dtype=jnp.int32)
        perm = ((p % G) * B + p // (L * G)) * L + (p // G) % L
    gidx = jnp.take(idx.reshape(-1), perm).reshape(NW, -1, 128)
    table = code_book.reshape(G * V, DC)
    code_vectors = _sc_gather(table, gidx, B, L)
    return code_vectors, perp


# speedup vs baseline: 1.0622x; 1.0622x over previous
import functools

import jax
import jax.numpy as jnp
from jax import lax
from jax.experimental import pallas as pl
from jax.experimental.pallas import tpu as pltpu
from jax.experimental.pallas import tpu_sc as plsc


def _tc_body(hs_ref, gum_ref, w_ref, b_ref, perp_ref, idx_ref, *, B, TL, G, V, gmin):
    D1 = hs_ref.shape[2]
    hs = hs_ref[...].reshape(B * TL, D1)
    logits = jnp.dot(hs, w_ref[...], preferred_element_type=jnp.float32)
    logits = logits + b_ref[...][None, :]
    perp_parts = []
    idx_parts = []
    for g in range(G):
        lg = logits[:, g * V:(g + 1) * V].reshape(B, TL, V)
        a = lg + gum_ref[:, :, g * V:(g + 1) * V]
        am = jnp.max(a, axis=-1, keepdims=True)
        ii = lax.broadcasted_iota(jnp.int32, (B, TL, V), 2)
        idx = jnp.min(jnp.where(a == am, ii, jnp.int32(2 ** 30)), axis=-1)
        idx_parts.append(idx + g * V)
        e = jnp.exp(lg - (am - gmin))
        s = jnp.sum(e, axis=-1, keepdims=True)
        p = jnp.sum(e / s, axis=0) * (1.0 / B)
        perp_parts.append(p[:, None, :])
    perp_ref[...] = jnp.concatenate(perp_parts, axis=1)
    idx_ref[...] = jnp.concatenate(idx_parts, axis=0)


def _tc_call(hs, gum, W, b2, TL):
    B, L, D1 = hs.shape
    GV = W.shape[1]
    G = 2
    V = GV // G
    grid = (L // TL,)
    return pl.pallas_call(
        functools.partial(_tc_body, B=B, TL=TL, G=G, V=V, gmin=-10.0),
        grid=grid,
        in_specs=[
            pl.BlockSpec((B, TL, D1), lambda i: (0, i, 0)),
            pl.BlockSpec((B, TL, GV), lambda i: (0, i, 0)),
            pl.BlockSpec((D1, GV), lambda i: (0, 0)),
            pl.BlockSpec((GV,), lambda i: (0,)),
        ],
        out_specs=[
            pl.BlockSpec((TL, G, V), lambda i: (i, 0, 0)),
            pl.BlockSpec((G * B, TL), lambda i: (0, i)),
        ],
        out_shape=[
            jax.ShapeDtypeStruct((L, G, V), jnp.float32),
            jax.ShapeDtypeStruct((G * B, L), jnp.int32),
        ],
        compiler_params=pltpu.CompilerParams(
            dimension_semantics=("parallel",)),
    )(hs, gum, W, b2)


def _sc_gather(table, gidx, B, L):
    info = plsc.get_sparse_core_info()
    NC, NS = info.num_cores, info.num_subcores
    NW = NC * NS
    CH = gidx.shape[1]
    DC = table.shape[1]
    G = NW * CH * 128 // (B * L)
    lw = L // (NW // B)
    wpb = NW // B
    mesh = plsc.VectorSubcoreMesh(core_axis_name="c", subcore_axis_name="s")

    @functools.partial(
        pl.kernel,
        mesh=mesh,
        out_type=jax.ShapeDtypeStruct((B, L, G * DC), jnp.float32),
        scratch_types=[
            pltpu.VMEM((CH, 128), jnp.int32),
            pltpu.VMEM((CH, 128, DC), jnp.float32),
        ] + [pltpu.SemaphoreType.DMA] * 5,
    )
    def k(table_hbm, idx_hbm, out_hbm, idx_v, rows_v, *sems):
        wid = lax.axis_index("s") * NC + lax.axis_index("c")
        wb = wid // wpb
        lworker = (wid % wpb) * lw
        lchunk = lw // CH
        pltpu.sync_copy(idx_hbm.at[wid], idx_v)
        gcopies = [
            pltpu.async_copy(table_hbm.at[idx_v.at[j]], rows_v.at[j], sems[j])
            for j in range(CH)
        ]
        wcopies = []
        for j in range(CH):
            gcopies[j].wait()
            wcopies.append(pltpu.async_copy(
                rows_v.at[j].reshape(lchunk, G * DC),
                out_hbm.at[wb, pl.ds(lworker + j * lchunk, lchunk), :],
                sems[CH]))
        for c in wcopies:
            c.wait()

    return k(table, gidx)


def kernel(hidden_states, W, b, code_book):
    B, L, D1 = hidden_states.shape
    _, G, V, DC = code_book.shape
    with jax.ensure_compile_time_eval():
        gum = jax.random.gumbel(jax.random.key(42), (B, L, G * V), jnp.float32)
    TL = 512
    perp, idx = _tc_call(hidden_states, gum, W, b, TL)
    info = plsc.get_sparse_core_info()
    NW = info.num_cores * info.num_subcores
    gidx = idx.reshape(G, B, L).transpose(1, 2, 0).reshape(NW, -1, 128)
    table = code_book.reshape(G * V, DC)
    code_vectors = _sc_gather(table, gidx, B, L)
    return code_vectors, perp


# speedup vs baseline: 1.1668x; 1.0984x over previous
import functools

import jax
import jax.numpy as jnp
from jax import lax
from jax.experimental import pallas as pl
from jax.experimental.pallas import tpu as pltpu
from jax.experimental.pallas import tpu_sc as plsc


def _tc_body(hs_ref, gum_ref, w_ref, b_ref, perp_ref, idx_ref, *, B, TL, G, V, gmin):
    D1 = hs_ref.shape[2]
    hs = hs_ref[...].reshape(B * TL, D1)
    logits = jnp.dot(hs, w_ref[...], preferred_element_type=jnp.float32)
    logits = logits + b_ref[...][None, :]
    perp_parts = []
    idx_parts = []
    for g in range(G):
        lg = logits[:, g * V:(g + 1) * V].reshape(B, TL, V)
        a = lg + gum_ref[:, :, g * V:(g + 1) * V]
        am = jnp.max(a, axis=-1, keepdims=True)
        ii = lax.broadcasted_iota(jnp.int32, (B, TL, V), 2)
        idx = jnp.min(jnp.where(a == am, ii, jnp.int32(2 ** 30)), axis=-1)
        idx_parts.append(idx + g * V)
        e = jnp.exp(lg - (am - gmin))
        s = jnp.sum(e, axis=-1, keepdims=True)
        p = jnp.sum(e / s, axis=0) * (1.0 / B)
        perp_parts.append(p)
    perp_ref[...] = jnp.concatenate(perp_parts, axis=-1)
    idx_ref[...] = jnp.concatenate(idx_parts, axis=0)


def _tc_call(hs, gum, W, b2, TL):
    B, L, D1 = hs.shape
    GV = W.shape[1]
    G = 2
    V = GV // G
    grid = (L // TL,)
    return pl.pallas_call(
        functools.partial(_tc_body, B=B, TL=TL, G=G, V=V, gmin=-10.0),
        grid=grid,
        in_specs=[
            pl.BlockSpec((B, TL, D1), lambda i: (0, i, 0)),
            pl.BlockSpec((B, TL, GV), lambda i: (0, i, 0)),
            pl.BlockSpec((D1, GV), lambda i: (0, 0)),
            pl.BlockSpec((GV,), lambda i: (0,)),
        ],
        out_specs=[
            pl.BlockSpec((TL, GV), lambda i: (i, 0)),
            pl.BlockSpec((G * B, TL), lambda i: (0, i)),
        ],
        out_shape=[
            jax.ShapeDtypeStruct((L, GV), jnp.float32),
            jax.ShapeDtypeStruct((G * B, L), jnp.int32),
        ],
        compiler_params=pltpu.CompilerParams(
            dimension_semantics=("parallel",)),
    )(hs, gum, W, b2)


def _sc_gather(table, gidx, B, L):
    info = plsc.get_sparse_core_info()
    NC, NS = info.num_cores, info.num_subcores
    NW = NC * NS
    CH = gidx.shape[1]
    DC = table.shape[1]
    G = NW * CH * 128 // (B * L)
    lw = L // (NW // B)
    wpb = NW // B
    mesh = plsc.VectorSubcoreMesh(core_axis_name="c", subcore_axis_name="s")

    @functools.partial(
        pl.kernel,
        mesh=mesh,
        out_type=jax.ShapeDtypeStruct((B, L, G * DC), jnp.float32),
        scratch_types=[
            pltpu.VMEM((CH, 128), jnp.int32),
            pltpu.VMEM((CH, 128, DC), jnp.float32),
        ] + [pltpu.SemaphoreType.DMA] * 5,
    )
    def k(table_hbm, idx_hbm, out_hbm, idx_v, rows_v, *sems):
        wid = lax.axis_index("s") * NC + lax.axis_index("c")
        wb = wid // wpb
        lworker = (wid % wpb) * lw
        lchunk = lw // CH
        pltpu.sync_copy(idx_hbm.at[wid], idx_v)
        gcopies = [
            pltpu.async_copy(table_hbm.at[idx_v.at[j]], rows_v.at[j], sems[j])
            for j in range(CH)
        ]
        wcopies = []
        for j in range(CH):
            gcopies[j].wait()
            wcopies.append(pltpu.async_copy(
                rows_v.at[j].reshape(lchunk, G * DC),
                out_hbm.at[wb, pl.ds(lworker + j * lchunk, lchunk), :],
                sems[CH]))
        for c in wcopies:
            c.wait()

    return k(table, gidx)


def kernel(hidden_states, W, b, code_book):
    B, L, D1 = hidden_states.shape
    _, G, V, DC = code_book.shape
    with jax.ensure_compile_time_eval():
        gum = jax.random.gumbel(jax.random.key(42), (B, L, G * V), jnp.float32)
    TL = 512
    perp, idx = _tc_call(hidden_states, gum, W, b, TL)
    perp = perp.reshape(L, G, V)
    info = plsc.get_sparse_core_info()
    NW = info.num_cores * info.num_subcores
    gidx = idx.reshape(G, B, L).transpose(1, 2, 0).reshape(NW, -1, 128)
    table = code_book.reshape(G * V, DC)
    code_vectors = _sc_gather(table, gidx, B, L)
    return code_vectors, perp
